# Initial kernel scaffold; baseline (speedup 1.0000x reference)
import jax
import jax.numpy as jnp
from jax.experimental import pallas as pl


def kernel(image, output, label, assign_2, assign_4):
    raise NotImplementedError("write your pallas kernel here")


# speedup vs baseline: 134.5135x; 134.5135x over previous
import functools

import jax
import jax.numpy as jnp
from jax import lax
from jax.experimental import pallas as pl
from jax.experimental.pallas import tpu as pltpu
from jax.experimental.pallas import tpu_sc as plsc

B = 4
H, W = 512, 512
H1, W1 = 256, 256
N1 = H1 * W1
S2 = 128 * 128
S4 = 64 * 64
NC, NS = 2, 16
PPS = N1 // NS
R2 = PPS // 128
R4 = (S2 // NS) // 128

_HIGH = lax.Precision.HIGHEST


def _feat_body(img_ref, out_ref):
    ii = lax.broadcasted_iota(jnp.int32, (H1, H), 0).astype(jnp.float32)
    jj = lax.broadcasted_iota(jnp.int32, (H1, H), 1).astype(jnp.float32)
    w = jnp.maximum(0.0, 1.0 - jnp.abs((jj - 2.0 * ii - 0.5) * 0.5))
    w = w / jnp.sum(w, axis=1, keepdims=True)

    def half(c):
        t = lax.dot(w, c, precision=_HIGH)
        return lax.dot_general(t, w, (((1,), (1,)), ((), ())),
                               precision=_HIGH)

    def to_linear(c):
        big = jnp.exp(2.4 * jnp.log(jnp.maximum((c + 0.055) * (1.0 / 1.055),
                                                1e-12)))
        return jnp.where(c <= 0.04045, c * (1.0 / 12.92), big)

    r = to_linear(half(img_ref[0, 0]))
    g = to_linear(half(img_ref[0, 1]))
    b = to_linear(half(img_ref[0, 2]))

    x = (0.412453 * r + 0.357580 * g + 0.180423 * b) * (1.0 / 0.950456)
    y = 0.212671 * r + 0.715160 * g + 0.072169 * b
    z = (0.019334 * r + 0.119193 * g + 0.950227 * b) * (1.0 / 1.088754)

    d = 6.0 / 29.0
    d3 = d * d * d

    def f(t):
        cbrt = jnp.exp(jnp.log(jnp.maximum(t, d3)) * (1.0 / 3.0))
        return jnp.where(t > d3, cbrt, t / (3.0 * d * d) + 4.0 / 29.0)

    fx, fy, fz = f(x), f(y), f(z)
    out_ref[0, 0] = 116.0 * fy - 16.0
    out_ref[0, 1] = 500.0 * (fx - fy)
    out_ref[0, 2] = 200.0 * (fy - fz)
    out_ref[0, 3] = lax.broadcasted_iota(jnp.int32, (H1, W1), 1).astype(
        jnp.float32)
    out_ref[0, 4] = lax.broadcasted_iota(jnp.int32, (H1, W1), 0).astype(
        jnp.float32)


_feat_kernel = pl.pallas_call(
    _feat_body,
    grid=(B,),
    in_specs=[pl.BlockSpec((1, 3, H, W), lambda b: (b, 0, 0, 0))],
    out_specs=pl.BlockSpec((1, 5, H1, W1), lambda b: (b, 0, 0, 0)),
    out_shape=jax.ShapeDtypeStruct((B, 5, H1, W1), jnp.float32),
)


def _sc_body(featsf, a2r, a4r, u,
             acc2, acc4,
             idx2, val2, idx4, val4, ones, cidx, gout, cbuf, dbuf, zbuf,
             a4full, m4buf):
    cc = lax.axis_index("c")
    ss = lax.axis_index("s")

    @pl.loop(0, 64)
    def _(k):
        zbuf[pl.ds(k * 16, 16)] = jnp.zeros((16,), jnp.float32)

    for k in range(8):
        ones[pl.ds(k * 16, 16)] = jnp.ones((16,), jnp.float32)

    for bb in range(2):
        b = cc * 2 + bb
        for ch in range(6):
            pltpu.sync_copy(zbuf, acc2.at[bb, ch, pl.ds(ss * 1024, 1024)])
            pltpu.sync_copy(zbuf.at[pl.ds(0, 256)],
                            acc4.at[bb, ch, pl.ds(ss * 256, 256)])
    plsc.subcore_barrier()

    for bb in range(2):
        b = cc * 2 + bb
        pltpu.sync_copy(a2r.at[b, pl.ds(ss * R2, R2)], idx2)

        @pl.loop(0, R2)
        def _(j):
            pltpu.sync_copy(ones, acc2.at[bb, 5].at[idx2.at[j]], add=True)

        for ch in range(5):
            pltpu.sync_copy(
                featsf.at[pl.ds((b * 5 + ch) * N1 + ss * PPS, PPS)], val2)

            @pl.loop(0, R2)
            def _(j):
                pltpu.sync_copy(val2.at[pl.ds(j * 128, 128)],
                                acc2.at[bb, ch].at[idx2.at[j]], add=True)
    plsc.subcore_barrier()

    for bb in range(2):
        pltpu.sync_copy(acc2.at[bb, 5, pl.ds(ss * 1024, 1024)], cbuf)
        for ch in range(5):
            pltpu.sync_copy(acc2.at[bb, ch, pl.ds(ss * 1024, 1024)], dbuf)

            @pl.loop(0, 64)
            def _(k):
                o = k * 16
                dbuf[pl.ds(o, 16)] = dbuf[pl.ds(o, 16)] / jnp.maximum(
                    cbuf[pl.ds(o, 16)], 1.0)

            pltpu.sync_copy(dbuf, acc2.at[bb, ch, pl.ds(ss * 1024, 1024)])
    plsc.subcore_barrier()

    for bb in range(2):
        b = cc * 2 + bb
        pltpu.sync_copy(a4r.at[b, pl.ds(ss * R4, R4)], idx4)

        @pl.loop(0, R4)
        def _(j):
            pltpu.sync_copy(ones, acc4.at[bb, 5].at[idx4.at[j]], add=True)

        for ch in range(5):
            pltpu.sync_copy(acc2.at[bb, ch, pl.ds(ss * 1024, 1024)], val4)

            @pl.loop(0, R4)
            def _(j):
                pltpu.sync_copy(val4.at[pl.ds(j * 128, 128)],
                                acc4.at[bb, ch].at[idx4.at[j]], add=True)
    plsc.subcore_barrier()

    for bb in range(2):
        pltpu.sync_copy(acc4.at[bb, 5, pl.ds(ss * 256, 256)],
                        cbuf.at[pl.ds(0, 256)])
        for ch in range(5):
            pltpu.sync_copy(acc4.at[bb, ch, pl.ds(ss * 256, 256)],
                            dbuf.at[pl.ds(0, 256)])

            @pl.loop(0, 16)
            def _(k):
                o = k * 16
                dbuf[pl.ds(o, 16)] = dbuf[pl.ds(o, 16)] / jnp.maximum(
                    cbuf[pl.ds(o, 16)], 1.0)

            pltpu.sync_copy(dbuf.at[pl.ds(0, 256)],
                            acc4.at[bb, ch, pl.ds(ss * 256, 256)])
    plsc.subcore_barrier()

    for bb in range(2):
        b = cc * 2 + bb
        pltpu.sync_copy(a2r.at[b, pl.ds(ss * R2, R2)], idx2)
        pltpu.sync_copy(a4r.at[b], a4full)

        @pl.loop(0, R2 * 8)
        def _(k):
            j = k // 8
            o = (k % 8) * 16
            a2v = idx2[j, pl.ds(o, 16)]
            cv = plsc.load_gather(a4full, [a2v >> 7, a2v & 127])
            cidx[j, pl.ds(o, 16)] = cv

        for ch in range(5):
            pltpu.sync_copy(acc4.at[bb, ch], m4buf)

            @pl.loop(0, R2 * 8)
            def _(k):
                j = k // 8
                o = (k % 8) * 16
                cv = cidx[j, pl.ds(o, 16)]
                gout[j, pl.ds(o, 16)] = plsc.load_gather(m4buf, [cv])

            pltpu.sync_copy(gout, u.at[b, ch, pl.ds(ss * R2, R2)])


_sc_kernel = functools.partial(
    pl.kernel,
    out_type=jax.ShapeDtypeStruct((B, 5, N1 // 128, 128), jnp.float32),
    mesh=plsc.VectorSubcoreMesh(core_axis_name="c", subcore_axis_name="s",
                                num_cores=NC, num_subcores=NS),
    compiler_params=pltpu.CompilerParams(use_tc_tiling_on_sc=False,
                                         needs_layout_passes=False),
    scratch_types=[
        pltpu.VMEM_SHARED((2, 6, S2), jnp.float32),
        pltpu.VMEM_SHARED((2, 6, S4), jnp.float32),
        pltpu.VMEM((R2, 128), jnp.int32),
        pltpu.VMEM((PPS,), jnp.float32),
        pltpu.VMEM((R4, 128), jnp.int32),
        pltpu.VMEM((1024,), jnp.float32),
        pltpu.VMEM((128,), jnp.float32),
        pltpu.VMEM((R2, 128), jnp.int32),
        pltpu.VMEM((R2, 128), jnp.float32),
        pltpu.VMEM((1024,), jnp.float32),
        pltpu.VMEM((1024,), jnp.float32),
        pltpu.VMEM((1024,), jnp.float32),
        pltpu.VMEM((S2 // 128, 128), jnp.int32),
        pltpu.VMEM((S4,), jnp.float32),
    ],
)(_sc_body)


def _loss_body(u_ref, f_ref, o1_ref, o2_ref):
    d0 = u_ref[0, 0] - f_ref[0, 0]
    d1 = u_ref[0, 1] - f_ref[0, 1]
    d2 = u_ref[0, 2] - f_ref[0, 2]
    d3 = u_ref[0, 3] - f_ref[0, 3]
    d4 = u_ref[0, 4] - f_ref[0, 4]
    s1 = jnp.sum(jnp.sqrt(d0 * d0 + d1 * d1 + d2 * d2))
    s2 = jnp.sum(jnp.sqrt(d3 * d3 + d4 * d4))

    @pl.when(pl.program_id(0) == 0)
    def _():
        o1_ref[...] = jnp.zeros((8, 128), jnp.float32)
        o2_ref[...] = jnp.zeros((8, 128), jnp.float32)

    o1_ref[...] += s1
    o2_ref[...] += s2


_loss_kernel = pl.pallas_call(
    _loss_body,
    grid=(B,),
    in_specs=[
        pl.BlockSpec((1, 5, N1 // 128, 128), lambda b: (b, 0, 0, 0)),
        pl.BlockSpec((1, 5, N1 // 128, 128), lambda b: (b, 0, 0, 0)),
    ],
    out_specs=[
        pl.BlockSpec((8, 128), lambda b: (0, 0)),
        pl.BlockSpec((8, 128), lambda b: (0, 0)),
    ],
    out_shape=[
        jax.ShapeDtypeStruct((8, 128), jnp.float32),
        jax.ShapeDtypeStruct((8, 128), jnp.float32),
    ],
)


def kernel(image, output, label, assign_2, assign_4):
    feats = _feat_kernel(image)
    featsf = feats.reshape(B, 5, N1)
    featsr = feats.reshape(B, 5, N1 // 128, 128)
    a2r = assign_2.reshape(B, N1 // 128, 128)
    a4r = assign_4.reshape(B, S2 // 128, 128)
    u = _sc_kernel(feats.reshape(-1), a2r, a4r)
    s1, s2 = _loss_kernel(u, featsr)
    npix = float(B * N1)
    return s1[0, 0] / npix + 0.1 * (s2[0, 0] / (npix * 16.0))


# speedup vs baseline: 171.2196x; 1.2729x over previous
import functools

import jax
import jax.numpy as jnp
from jax import lax
from jax.experimental import pallas as pl
from jax.experimental.pallas import tpu as pltpu
from jax.experimental.pallas import tpu_sc as plsc

B = 4
H, W = 512, 512
H1, W1 = 256, 256
N1 = H1 * W1
S2 = 128 * 128
S4 = 64 * 64
NC, NS = 2, 16
PPS = N1 // NS
R2 = PPS // 128
R4 = (S2 // NS) // 128

_HIGH = lax.Precision.HIGHEST


def _feat_body(img_ref, out_ref):
    ii = lax.broadcasted_iota(jnp.int32, (H1, H), 0).astype(jnp.float32)
    jj = lax.broadcasted_iota(jnp.int32, (H1, H), 1).astype(jnp.float32)
    w = jnp.maximum(0.0, 1.0 - jnp.abs((jj - 2.0 * ii - 0.5) * 0.5))
    w = w / jnp.sum(w, axis=1, keepdims=True)

    def half(c):
        t = lax.dot(w, c, precision=_HIGH)
        return lax.dot_general(t, w, (((1,), (1,)), ((), ())),
                               precision=_HIGH)

    def to_linear(c):
        big = jnp.exp(2.4 * jnp.log(jnp.maximum((c + 0.055) * (1.0 / 1.055),
                                                1e-12)))
        return jnp.where(c <= 0.04045, c * (1.0 / 12.92), big)

    r = to_linear(half(img_ref[0, 0]))
    g = to_linear(half(img_ref[0, 1]))
    b = to_linear(half(img_ref[0, 2]))

    x = (0.412453 * r + 0.357580 * g + 0.180423 * b) * (1.0 / 0.950456)
    y = 0.212671 * r + 0.715160 * g + 0.072169 * b
    z = (0.019334 * r + 0.119193 * g + 0.950227 * b) * (1.0 / 1.088754)

    d = 6.0 / 29.0
    d3 = d * d * d

    def f(t):
        cbrt = jnp.exp(jnp.log(jnp.maximum(t, d3)) * (1.0 / 3.0))
        return jnp.where(t > d3, cbrt, t / (3.0 * d * d) + 4.0 / 29.0)

    fx, fy, fz = f(x), f(y), f(z)
    out_ref[0, 0] = 116.0 * fy - 16.0
    out_ref[0, 1] = 500.0 * (fx - fy)
    out_ref[0, 2] = 200.0 * (fy - fz)
    out_ref[0, 3] = lax.broadcasted_iota(jnp.int32, (H1, W1), 1).astype(
        jnp.float32)
    out_ref[0, 4] = lax.broadcasted_iota(jnp.int32, (H1, W1), 0).astype(
        jnp.float32)


_feat_kernel = pl.pallas_call(
    _feat_body,
    grid=(B,),
    in_specs=[pl.BlockSpec((1, 3, H, W), lambda b: (b, 0, 0, 0))],
    out_specs=pl.BlockSpec((1, 5, H1, W1), lambda b: (b, 0, 0, 0)),
    out_shape=jax.ShapeDtypeStruct((B, 5, H1, W1), jnp.float32),
)


def _sc_body(featsf, a2r, a4r, u,
             acc2, acc4,
             idx2, val25, idx4, val45, ones, cidx, gout, cbuf, dbuf, zbuf,
             a4full, m4buf, sem):
    cc = lax.axis_index("c")
    ss = lax.axis_index("s")

    @pl.loop(0, 64)
    def _(k):
        zbuf[pl.ds(k * 16, 16)] = jnp.zeros((16,), jnp.float32)

    for k in range(8):
        ones[pl.ds(k * 16, 16)] = jnp.ones((16,), jnp.float32)

    for bb in range(2):
        b = cc * 2 + bb
        for ch in range(6):
            pltpu.sync_copy(zbuf, acc2.at[bb, ch, pl.ds(ss * 1024, 1024)])
            pltpu.sync_copy(zbuf.at[pl.ds(0, 256)],
                            acc4.at[bb, ch, pl.ds(ss * 256, 256)])
    plsc.subcore_barrier()

    for bb in range(2):
        b = cc * 2 + bb
        pltpu.sync_copy(a2r.at[b, pl.ds(ss * R2, R2)], idx2)
        for ch in range(5):
            pltpu.sync_copy(
                featsf.at[pl.ds((b * 5 + ch) * N1 + ss * PPS, PPS)],
                val25.at[ch])

        @pl.loop(0, R2)
        def _(j):
            pltpu.async_copy(ones, acc2.at[bb, 5].at[idx2.at[j]], sem,
                             add=True)
            for ch in range(5):
                pltpu.async_copy(val25.at[ch, pl.ds(j * 128, 128)],
                                 acc2.at[bb, ch].at[idx2.at[j]], sem,
                                 add=True)

        for _k in range(6):
            pltpu.make_async_copy(featsf.at[pl.ds(0, PPS)],
                                  val25.at[0], sem).wait()
    plsc.subcore_barrier()

    for bb in range(2):
        pltpu.sync_copy(acc2.at[bb, 5, pl.ds(ss * 1024, 1024)], cbuf)
        for ch in range(5):
            pltpu.sync_copy(acc2.at[bb, ch, pl.ds(ss * 1024, 1024)], dbuf)

            @pl.loop(0, 64)
            def _(k):
                o = k * 16
                dbuf[pl.ds(o, 16)] = dbuf[pl.ds(o, 16)] / jnp.maximum(
                    cbuf[pl.ds(o, 16)], 1.0)

            pltpu.sync_copy(dbuf, acc2.at[bb, ch, pl.ds(ss * 1024, 1024)])
    plsc.subcore_barrier()

    for bb in range(2):
        b = cc * 2 + bb
        pltpu.sync_copy(a4r.at[b, pl.ds(ss * R4, R4)], idx4)
        for ch in range(5):
            pltpu.sync_copy(acc2.at[bb, ch, pl.ds(ss * 1024, 1024)],
                            val45.at[ch])

        @pl.loop(0, R4)
        def _(j):
            pltpu.async_copy(ones, acc4.at[bb, 5].at[idx4.at[j]], sem,
                             add=True)
            for ch in range(5):
                pltpu.async_copy(val45.at[ch, pl.ds(j * 128, 128)],
                                 acc4.at[bb, ch].at[idx4.at[j]], sem,
                                 add=True)

        for _k in range(6):
            pltpu.make_async_copy(featsf.at[pl.ds(0, 1024)],
                                  val45.at[0], sem).wait()
    plsc.subcore_barrier()

    for bb in range(2):
        pltpu.sync_copy(acc4.at[bb, 5, pl.ds(ss * 256, 256)],
                        cbuf.at[pl.ds(0, 256)])
        for ch in range(5):
            pltpu.sync_copy(acc4.at[bb, ch, pl.ds(ss * 256, 256)],
                            dbuf.at[pl.ds(0, 256)])

            @pl.loop(0, 16)
            def _(k):
                o = k * 16
                dbuf[pl.ds(o, 16)] = dbuf[pl.ds(o, 16)] / jnp.maximum(
                    cbuf[pl.ds(o, 16)], 1.0)

            pltpu.sync_copy(dbuf.at[pl.ds(0, 256)],
                            acc4.at[bb, ch, pl.ds(ss * 256, 256)])
    plsc.subcore_barrier()

    for bb in range(2):
        b = cc * 2 + bb
        pltpu.sync_copy(a2r.at[b, pl.ds(ss * R2, R2)], idx2)
        pltpu.sync_copy(a4r.at[b], a4full)

        @pl.loop(0, R2)
        def _(j):
            for l in range(8):
                o = l * 16
                a2v = idx2[j, pl.ds(o, 16)]
                cv = plsc.load_gather(a4full, [a2v >> 7, a2v & 127])
                cidx[j, pl.ds(o, 16)] = cv

        for ch in range(5):
            pltpu.sync_copy(acc4.at[bb, ch], m4buf)

            @pl.loop(0, R2)
            def _(j):
                for l in range(8):
                    o = l * 16
                    cv = cidx[j, pl.ds(o, 16)]
                    gout[j, pl.ds(o, 16)] = plsc.load_gather(m4buf, [cv])

            pltpu.sync_copy(gout, u.at[b, ch, pl.ds(ss * R2, R2)])


_sc_kernel = functools.partial(
    pl.kernel,
    out_type=jax.ShapeDtypeStruct((B, 5, N1 // 128, 128), jnp.float32),
    mesh=plsc.VectorSubcoreMesh(core_axis_name="c", subcore_axis_name="s",
                                num_cores=NC, num_subcores=NS),
    compiler_params=pltpu.CompilerParams(use_tc_tiling_on_sc=False,
                                         needs_layout_passes=False),
    scratch_types=[
        pltpu.VMEM_SHARED((2, 6, S2), jnp.float32),
        pltpu.VMEM_SHARED((2, 6, S4), jnp.float32),
        pltpu.VMEM((R2, 128), jnp.int32),
        pltpu.VMEM((5, PPS), jnp.float32),
        pltpu.VMEM((R4, 128), jnp.int32),
        pltpu.VMEM((5, 1024), jnp.float32),
        pltpu.VMEM((128,), jnp.float32),
        pltpu.VMEM((R2, 128), jnp.int32),
        pltpu.VMEM((R2, 128), jnp.float32),
        pltpu.VMEM((1024,), jnp.float32),
        pltpu.VMEM((1024,), jnp.float32),
        pltpu.VMEM((1024,), jnp.float32),
        pltpu.VMEM((S2 // 128, 128), jnp.int32),
        pltpu.VMEM((S4,), jnp.float32),
        pltpu.SemaphoreType.DMA,
    ],
)(_sc_body)


def _loss_body(u_ref, f_ref, o1_ref, o2_ref):
    d0 = u_ref[0, 0] - f_ref[0, 0]
    d1 = u_ref[0, 1] - f_ref[0, 1]
    d2 = u_ref[0, 2] - f_ref[0, 2]
    d3 = u_ref[0, 3] - f_ref[0, 3]
    d4 = u_ref[0, 4] - f_ref[0, 4]
    s1 = jnp.sum(jnp.sqrt(d0 * d0 + d1 * d1 + d2 * d2))
    s2 = jnp.sum(jnp.sqrt(d3 * d3 + d4 * d4))

    @pl.when(pl.program_id(0) == 0)
    def _():
        o1_ref[...] = jnp.zeros((8, 128), jnp.float32)
        o2_ref[...] = jnp.zeros((8, 128), jnp.float32)

    o1_ref[...] += s1
    o2_ref[...] += s2


_loss_kernel = pl.pallas_call(
    _loss_body,
    grid=(B,),
    in_specs=[
        pl.BlockSpec((1, 5, N1 // 128, 128), lambda b: (b, 0, 0, 0)),
        pl.BlockSpec((1, 5, N1 // 128, 128), lambda b: (b, 0, 0, 0)),
    ],
    out_specs=[
        pl.BlockSpec((8, 128), lambda b: (0, 0)),
        pl.BlockSpec((8, 128), lambda b: (0, 0)),
    ],
    out_shape=[
        jax.ShapeDtypeStruct((8, 128), jnp.float32),
        jax.ShapeDtypeStruct((8, 128), jnp.float32),
    ],
)


def kernel(image, output, label, assign_2, assign_4):
    feats = _feat_kernel(image)
    featsf = feats.reshape(B, 5, N1)
    featsr = feats.reshape(B, 5, N1 // 128, 128)
    a2r = assign_2.reshape(B, N1 // 128, 128)
    a4r = assign_4.reshape(B, S2 // 128, 128)
    u = _sc_kernel(feats.reshape(-1), a2r, a4r)
    s1, s2 = _loss_kernel(u, featsr)
    npix = float(B * N1)
    return s1[0, 0] / npix + 0.1 * (s2[0, 0] / (npix * 16.0))


# speedup vs baseline: 177.6682x; 1.0377x over previous
import functools

import jax
import jax.numpy as jnp
from jax import lax
from jax.experimental import pallas as pl
from jax.experimental.pallas import tpu as pltpu
from jax.experimental.pallas import tpu_sc as plsc

B = 4
H, W = 512, 512
H1, W1 = 256, 256
N1 = H1 * W1
S2 = 128 * 128
S4 = 64 * 64
NC, NS = 2, 16
PPS = N1 // NS
R2 = PPS // 128
R4 = (S2 // NS) // 128

_HIGH = lax.Precision.HIGHEST


def _feat_body(img_ref, out_ref):
    ii = lax.broadcasted_iota(jnp.int32, (H1, H), 0).astype(jnp.float32)
    jj = lax.broadcasted_iota(jnp.int32, (H1, H), 1).astype(jnp.float32)
    w = jnp.maximum(0.0, 1.0 - jnp.abs((jj - 2.0 * ii - 0.5) * 0.5))
    w = w / jnp.sum(w, axis=1, keepdims=True)

    def half(c):
        t = lax.dot(w, c, precision=_HIGH)
        return lax.dot_general(t, w, (((1,), (1,)), ((), ())),
                               precision=_HIGH)

    def to_linear(c):
        big = jnp.exp(2.4 * jnp.log(jnp.maximum((c + 0.055) * (1.0 / 1.055),
                                                1e-12)))
        return jnp.where(c <= 0.04045, c * (1.0 / 12.92), big)

    r = to_linear(half(img_ref[0, 0]))
    g = to_linear(half(img_ref[0, 1]))
    b = to_linear(half(img_ref[0, 2]))

    x = (0.412453 * r + 0.357580 * g + 0.180423 * b) * (1.0 / 0.950456)
    y = 0.212671 * r + 0.715160 * g + 0.072169 * b
    z = (0.019334 * r + 0.119193 * g + 0.950227 * b) * (1.0 / 1.088754)

    d = 6.0 / 29.0
    d3 = d * d * d

    def f(t):
        cbrt = jnp.exp(jnp.log(jnp.maximum(t, d3)) * (1.0 / 3.0))
        return jnp.where(t > d3, cbrt, t / (3.0 * d * d) + 4.0 / 29.0)

    fx, fy, fz = f(x), f(y), f(z)
    out_ref[0, 0] = 116.0 * fy - 16.0
    out_ref[0, 1] = 500.0 * (fx - fy)
    out_ref[0, 2] = 200.0 * (fy - fz)
    out_ref[0, 3] = lax.broadcasted_iota(jnp.int32, (H1, W1), 1).astype(
        jnp.float32)
    out_ref[0, 4] = lax.broadcasted_iota(jnp.int32, (H1, W1), 0).astype(
        jnp.float32)


_feat_kernel = pl.pallas_call(
    _feat_body,
    grid=(B,),
    in_specs=[pl.BlockSpec((1, 3, H, W), lambda b: (b, 0, 0, 0))],
    out_specs=pl.BlockSpec((1, 5, H1, W1), lambda b: (b, 0, 0, 0)),
    out_shape=jax.ShapeDtypeStruct((B, 5, H1, W1), jnp.float32),
)


def _sc_body(featsf, a2f, a4f, u,
             acc2, acc4, m4p, a4t,
             a2c, a4c, a4c2, val25, val45, gbuf, ones, cbuf, dbuf, zbuf, sem):
    cc = lax.axis_index("c")
    ss = lax.axis_index("s")

    @pl.loop(0, 64)
    def _(k):
        zbuf[pl.ds(k * 16, 16)] = jnp.zeros((16,), jnp.float32)

    for k in range(8):
        ones[pl.ds(k * 16, 16)] = jnp.ones((16,), jnp.float32)

    for bb in range(2):
        b = cc * 2 + bb
        for ch in range(6):
            pltpu.sync_copy(zbuf,
                            acc2.at[pl.ds((bb * 6 + ch) * S2 + ss * 1024,
                                          1024)])
            pltpu.sync_copy(zbuf.at[pl.ds(0, 256)],
                            acc4.at[pl.ds((bb * 6 + ch) * S4 + ss * 256,
                                          256)])
        pltpu.sync_copy(a4f.at[pl.ds(b * S2 + ss * 1024, 1024)],
                        a4t.at[pl.ds(bb * S2 + ss * 1024, 1024)])
    plsc.subcore_barrier()

    for bb in range(2):
        b = cc * 2 + bb
        pltpu.sync_copy(a2f.at[pl.ds(b * N1 + ss * PPS, PPS)], a2c)
        for ch in range(5):
            pltpu.sync_copy(
                featsf.at[pl.ds((b * 5 + ch) * N1 + ss * PPS, PPS)],
                val25.at[ch])

        @pl.loop(0, R2)
        def _(j):
            ix = a2c.at[pl.ds(j * 128, 128)]
            pltpu.async_copy(ones, acc2.at[pl.ds(bb * 6 * S2 + 5 * S2,
                                                 S2)].at[ix], sem, add=True)
            for ch in range(5):
                pltpu.async_copy(val25.at[ch, pl.ds(j * 128, 128)],
                                 acc2.at[pl.ds((bb * 6 + ch) * S2,
                                               S2)].at[ix], sem, add=True)

        for _k in range(6):
            pltpu.make_async_copy(featsf.at[pl.ds(0, PPS)],
                                  val25.at[0], sem).wait()
    plsc.subcore_barrier()

    for bb in range(2):
        pltpu.sync_copy(acc2.at[pl.ds(bb * 6 * S2 + 5 * S2 + ss * 1024,
                                      1024)], cbuf)
        for ch in range(5):
            o2 = (bb * 6 + ch) * S2 + ss * 1024
            pltpu.sync_copy(acc2.at[pl.ds(o2, 1024)], dbuf)

            @pl.loop(0, 64)
            def _(k):
                o = k * 16
                dbuf[pl.ds(o, 16)] = dbuf[pl.ds(o, 16)] / jnp.maximum(
                    cbuf[pl.ds(o, 16)], 1.0)

            pltpu.sync_copy(dbuf, acc2.at[pl.ds(o2, 1024)])
    plsc.subcore_barrier()

    for bb in range(2):
        b = cc * 2 + bb
        pltpu.sync_copy(a4f.at[pl.ds(b * S2 + ss * 1024, 1024)], a4c)
        for ch in range(5):
            pltpu.sync_copy(acc2.at[pl.ds((bb * 6 + ch) * S2 + ss * 1024,
                                          1024)], val45.at[ch])

        @pl.loop(0, R4)
        def _(j):
            ix = a4c.at[pl.ds(j * 128, 128)]
            pltpu.async_copy(ones, acc4.at[pl.ds(bb * 6 * S4 + 5 * S4,
                                                 S4)].at[ix], sem, add=True)
            for ch in range(5):
                pltpu.async_copy(val45.at[ch, pl.ds(j * 128, 128)],
                                 acc4.at[pl.ds((bb * 6 + ch) * S4,
                                               S4)].at[ix], sem, add=True)

        for _k in range(6):
            pltpu.make_async_copy(featsf.at[pl.ds(0, 1024)],
                                  val45.at[0], sem).wait()
    plsc.subcore_barrier()

    for bb in range(2):
        pltpu.sync_copy(acc4.at[pl.ds(bb * 6 * S4 + 5 * S4 + ss * 256,
                                      256)], cbuf.at[pl.ds(0, 256)])
        for ch in range(5):
            pltpu.sync_copy(acc4.at[pl.ds((bb * 6 + ch) * S4 + ss * 256,
                                          256)], dbuf.at[pl.ds(0, 256)])

            @pl.loop(0, 16)
            def _(k):
                o = k * 16
                dbuf[pl.ds(o, 16)] = dbuf[pl.ds(o, 16)] / jnp.maximum(
                    cbuf[pl.ds(o, 16)], 1.0)

            pltpu.sync_copy(dbuf.at[pl.ds(0, 256)],
                            m4p.at[pl.ds((bb * 5 + ch) * S4 + ss * 256,
                                         256)])
    plsc.subcore_barrier()

    for bb in range(2):
        b = cc * 2 + bb
        pltpu.sync_copy(a2f.at[pl.ds(b * N1 + ss * PPS, PPS)], a2c)
        pltpu.sync_copy(a4t.at[pl.ds(bb * S2, S2)].at[a2c], a4c2)

        for ch in range(5):
            pltpu.async_copy(m4p.at[pl.ds((bb * 5 + ch) * S4,
                                          S4)].at[a4c2], gbuf.at[ch], sem)
        for _k in range(5):
            pltpu.make_async_copy(featsf.at[pl.ds(0, PPS)],
                                  gbuf.at[0], sem).wait()

        for ch in range(5):
            pltpu.async_copy(gbuf.at[ch],
                             u.at[pl.ds((b * 5 + ch) * N1 + ss * PPS, PPS)],
                             sem)
        for _k in range(5):
            pltpu.make_async_copy(featsf.at[pl.ds(0, PPS)],
                                  gbuf.at[0], sem).wait()


_sc_kernel = functools.partial(
    pl.kernel,
    out_type=jax.ShapeDtypeStruct((B * 5 * N1,), jnp.float32),
    mesh=plsc.VectorSubcoreMesh(core_axis_name="c", subcore_axis_name="s",
                                num_cores=NC, num_subcores=NS),
    compiler_params=pltpu.CompilerParams(use_tc_tiling_on_sc=False,
                                         needs_layout_passes=False),
    scratch_types=[
        pltpu.VMEM_SHARED((2 * 6 * S2,), jnp.float32),
        pltpu.VMEM_SHARED((2 * 6 * S4,), jnp.float32),
        pltpu.VMEM_SHARED((2 * 5 * S4,), jnp.float32),
        pltpu.VMEM_SHARED((2 * S2,), jnp.int32),
        pltpu.VMEM((PPS,), jnp.int32),
        pltpu.VMEM((1024,), jnp.int32),
        pltpu.VMEM((PPS,), jnp.int32),
        pltpu.VMEM((5, PPS), jnp.float32),
        pltpu.VMEM((5, 1024), jnp.float32),
        pltpu.VMEM((5, PPS), jnp.float32),
        pltpu.VMEM((128,), jnp.float32),
        pltpu.VMEM((1024,), jnp.float32),
        pltpu.VMEM((1024,), jnp.float32),
        pltpu.VMEM((1024,), jnp.float32),
        pltpu.SemaphoreType.DMA,
    ],
)(_sc_body)


def _loss_body(u_ref, f_ref, o1_ref, o2_ref):
    d0 = u_ref[0, 0] - f_ref[0, 0]
    d1 = u_ref[0, 1] - f_ref[0, 1]
    d2 = u_ref[0, 2] - f_ref[0, 2]
    d3 = u_ref[0, 3] - f_ref[0, 3]
    d4 = u_ref[0, 4] - f_ref[0, 4]
    s1 = jnp.sum(jnp.sqrt(d0 * d0 + d1 * d1 + d2 * d2))
    s2 = jnp.sum(jnp.sqrt(d3 * d3 + d4 * d4))

    @pl.when(pl.program_id(0) == 0)
    def _():
        o1_ref[...] = jnp.zeros((8, 128), jnp.float32)
        o2_ref[...] = jnp.zeros((8, 128), jnp.float32)

    o1_ref[...] += s1
    o2_ref[...] += s2


_loss_kernel = pl.pallas_call(
    _loss_body,
    grid=(B,),
    in_specs=[
        pl.BlockSpec((1, 5, N1 // 128, 128), lambda b: (b, 0, 0, 0)),
        pl.BlockSpec((1, 5, N1 // 128, 128), lambda b: (b, 0, 0, 0)),
    ],
    out_specs=[
        pl.BlockSpec((8, 128), lambda b: (0, 0)),
        pl.BlockSpec((8, 128), lambda b: (0, 0)),
    ],
    out_shape=[
        jax.ShapeDtypeStruct((8, 128), jnp.float32),
        jax.ShapeDtypeStruct((8, 128), jnp.float32),
    ],
)


def kernel(image, output, label, assign_2, assign_4):
    feats = _feat_kernel(image)
    featsf = feats.reshape(B, 5, N1)
    featsr = feats.reshape(B, 5, N1 // 128, 128)
    uf = _sc_kernel(feats.reshape(-1), assign_2.reshape(-1),
                    assign_4.reshape(-1))
    u = uf.reshape(B, 5, N1 // 128, 128)
    s1, s2 = _loss_kernel(u, featsr)
    npix = float(B * N1)
    return s1[0, 0] / npix + 0.1 * (s2[0, 0] / (npix * 16.0))


# speedup vs baseline: 199.0875x; 1.1206x over previous
import functools

import jax
import jax.numpy as jnp
from jax import lax
from jax.experimental import pallas as pl
from jax.experimental.pallas import tpu as pltpu
from jax.experimental.pallas import tpu_sc as plsc

B = 4
H, W = 512, 512
H1, W1 = 256, 256
N1 = H1 * W1
S2 = 128 * 128
S4 = 64 * 64
NC, NS = 2, 16
PPS = N1 // NS
R2 = PPS // 128
R4 = (S2 // NS) // 128

_HIGH = lax.Precision.HIGHEST


def _feat_body(img_ref, out_ref):
    ii = lax.broadcasted_iota(jnp.int32, (H1, H), 0).astype(jnp.float32)
    jj = lax.broadcasted_iota(jnp.int32, (H1, H), 1).astype(jnp.float32)
    w = jnp.maximum(0.0, 1.0 - jnp.abs((jj - 2.0 * ii - 0.5) * 0.5))
    w = w / jnp.sum(w, axis=1, keepdims=True)

    def half(c):
        t = lax.dot(w, c)
        return lax.dot_general(t, w, (((1,), (1,)), ((), ())))

    def to_linear(c):
        big = jnp.exp(2.4 * jnp.log(jnp.maximum((c + 0.055) * (1.0 / 1.055),
                                                1e-12)))
        return jnp.where(c <= 0.04045, c * (1.0 / 12.92), big)

    r = to_linear(half(img_ref[0, 0]))
    g = to_linear(half(img_ref[0, 1]))
    b = to_linear(half(img_ref[0, 2]))

    x = (0.412453 * r + 0.357580 * g + 0.180423 * b) * (1.0 / 0.950456)
    y = 0.212671 * r + 0.715160 * g + 0.072169 * b
    z = (0.019334 * r + 0.119193 * g + 0.950227 * b) * (1.0 / 1.088754)

    d = 6.0 / 29.0
    d3 = d * d * d

    def f(t):
        cbrt = jnp.exp(jnp.log(jnp.maximum(t, d3)) * (1.0 / 3.0))
        return jnp.where(t > d3, cbrt, t / (3.0 * d * d) + 4.0 / 29.0)

    fx, fy, fz = f(x), f(y), f(z)
    out_ref[0, 0] = 116.0 * fy - 16.0
    out_ref[0, 1] = 500.0 * (fx - fy)
    out_ref[0, 2] = 200.0 * (fy - fz)
    out_ref[0, 3] = lax.broadcasted_iota(jnp.int32, (H1, W1), 1).astype(
        jnp.float32)
    out_ref[0, 4] = lax.broadcasted_iota(jnp.int32, (H1, W1), 0).astype(
        jnp.float32)


_feat_kernel = pl.pallas_call(
    _feat_body,
    grid=(B,),
    in_specs=[pl.BlockSpec((1, 3, H, W), lambda b: (b, 0, 0, 0))],
    out_specs=pl.BlockSpec((1, 5, H1, W1), lambda b: (b, 0, 0, 0)),
    out_shape=jax.ShapeDtypeStruct((B, 5, H1, W1), jnp.float32),
)


def _sc_body(featsf, a2f, a4f, u,
             acc2, acc4, m4p, a4t,
             a2c, a4c, a4c2, val25, val45, gbuf, ones, cbuf, dbuf, zbuf, sem):
    cc = lax.axis_index("c")
    ss = lax.axis_index("s")

    @pl.loop(0, 64)
    def _(k):
        zbuf[pl.ds(k * 16, 16)] = jnp.zeros((16,), jnp.float32)

    for k in range(8):
        ones[pl.ds(k * 16, 16)] = jnp.ones((16,), jnp.float32)

    for bb in range(2):
        b = cc * 2 + bb
        for ch in range(6):
            pltpu.sync_copy(zbuf,
                            acc2.at[pl.ds((bb * 6 + ch) * S2 + ss * 1024,
                                          1024)])
            pltpu.sync_copy(zbuf.at[pl.ds(0, 256)],
                            acc4.at[pl.ds((bb * 6 + ch) * S4 + ss * 256,
                                          256)])
        pltpu.sync_copy(a4f.at[pl.ds(b * S2 + ss * 1024, 1024)],
                        a4t.at[pl.ds(bb * S2 + ss * 1024, 1024)])
    plsc.subcore_barrier()

    for bb in range(2):
        b = cc * 2 + bb
        pltpu.sync_copy(a2f.at[pl.ds(b * N1 + ss * PPS, PPS)], a2c)
        for ch in range(5):
            pltpu.sync_copy(
                featsf.at[pl.ds((b * 5 + ch) * N1 + ss * PPS, PPS)],
                val25.at[ch])

        @pl.loop(0, R2)
        def _(j):
            ix = a2c.at[pl.ds(j * 128, 128)]
            pltpu.async_copy(ones, acc2.at[pl.ds(bb * 6 * S2 + 5 * S2,
                                                 S2)].at[ix], sem, add=True)
            for ch in range(5):
                pltpu.async_copy(val25.at[ch, pl.ds(j * 128, 128)],
                                 acc2.at[pl.ds((bb * 6 + ch) * S2,
                                               S2)].at[ix], sem, add=True)

        for _k in range(6):
            pltpu.make_async_copy(featsf.at[pl.ds(0, PPS)],
                                  val25.at[0], sem).wait()
    plsc.subcore_barrier()

    for bb in range(2):
        pltpu.sync_copy(acc2.at[pl.ds(bb * 6 * S2 + 5 * S2 + ss * 1024,
                                      1024)], cbuf)
        for ch in range(5):
            o2 = (bb * 6 + ch) * S2 + ss * 1024
            pltpu.sync_copy(acc2.at[pl.ds(o2, 1024)], dbuf)

            @pl.loop(0, 64)
            def _(k):
                o = k * 16
                dbuf[pl.ds(o, 16)] = dbuf[pl.ds(o, 16)] / jnp.maximum(
                    cbuf[pl.ds(o, 16)], 1.0)

            pltpu.sync_copy(dbuf, acc2.at[pl.ds(o2, 1024)])
    plsc.subcore_barrier()

    for bb in range(2):
        b = cc * 2 + bb
        pltpu.sync_copy(a4f.at[pl.ds(b * S2 + ss * 1024, 1024)], a4c)
        for ch in range(5):
            pltpu.sync_copy(acc2.at[pl.ds((bb * 6 + ch) * S2 + ss * 1024,
                                          1024)], val45.at[ch])

        @pl.loop(0, R4)
        def _(j):
            ix = a4c.at[pl.ds(j * 128, 128)]
            pltpu.async_copy(ones, acc4.at[pl.ds(bb * 6 * S4 + 5 * S4,
                                                 S4)].at[ix], sem, add=True)
            for ch in range(5):
                pltpu.async_copy(val45.at[ch, pl.ds(j * 128, 128)],
                                 acc4.at[pl.ds((bb * 6 + ch) * S4,
                                               S4)].at[ix], sem, add=True)

        for _k in range(6):
            pltpu.make_async_copy(featsf.at[pl.ds(0, 1024)],
                                  val45.at[0], sem).wait()
    plsc.subcore_barrier()

    for bb in range(2):
        pltpu.sync_copy(acc4.at[pl.ds(bb * 6 * S4 + 5 * S4 + ss * 256,
                                      256)], cbuf.at[pl.ds(0, 256)])
        for ch in range(5):
            pltpu.sync_copy(acc4.at[pl.ds((bb * 6 + ch) * S4 + ss * 256,
                                          256)], dbuf.at[pl.ds(0, 256)])

            @pl.loop(0, 16)
            def _(k):
                o = k * 16
                dbuf[pl.ds(o, 16)] = dbuf[pl.ds(o, 16)] / jnp.maximum(
                    cbuf[pl.ds(o, 16)], 1.0)

            pltpu.sync_copy(dbuf.at[pl.ds(0, 256)],
                            m4p.at[pl.ds((bb * 5 + ch) * S4 + ss * 256,
                                         256)])
    plsc.subcore_barrier()

    for bb in range(2):
        b = cc * 2 + bb
        pltpu.sync_copy(a2f.at[pl.ds(b * N1 + ss * PPS, PPS)], a2c)
        pltpu.sync_copy(a4t.at[pl.ds(bb * S2, S2)].at[a2c], a4c2)

        for ch in range(5):
            pltpu.async_copy(m4p.at[pl.ds((bb * 5 + ch) * S4,
                                          S4)].at[a4c2], gbuf.at[ch], sem)
        for _k in range(5):
            pltpu.make_async_copy(featsf.at[pl.ds(0, PPS)],
                                  gbuf.at[0], sem).wait()

        for ch in range(5):
            pltpu.async_copy(gbuf.at[ch],
                             u.at[pl.ds((b * 5 + ch) * N1 + ss * PPS, PPS)],
                             sem)
        for _k in range(5):
            pltpu.make_async_copy(featsf.at[pl.ds(0, PPS)],
                                  gbuf.at[0], sem).wait()


_sc_kernel = functools.partial(
    pl.kernel,
    out_type=jax.ShapeDtypeStruct((B * 5 * N1,), jnp.float32),
    mesh=plsc.VectorSubcoreMesh(core_axis_name="c", subcore_axis_name="s",
                                num_cores=NC, num_subcores=NS),
    compiler_params=pltpu.CompilerParams(use_tc_tiling_on_sc=False,
                                         needs_layout_passes=False),
    scratch_types=[
        pltpu.VMEM_SHARED((2 * 6 * S2,), jnp.float32),
        pltpu.VMEM_SHARED((2 * 6 * S4,), jnp.float32),
        pltpu.VMEM_SHARED((2 * 5 * S4,), jnp.float32),
        pltpu.VMEM_SHARED((2 * S2,), jnp.int32),
        pltpu.VMEM((PPS,), jnp.int32),
        pltpu.VMEM((1024,), jnp.int32),
        pltpu.VMEM((PPS,), jnp.int32),
        pltpu.VMEM((5, PPS), jnp.float32),
        pltpu.VMEM((5, 1024), jnp.float32),
        pltpu.VMEM((5, PPS), jnp.float32),
        pltpu.VMEM((128,), jnp.float32),
        pltpu.VMEM((1024,), jnp.float32),
        pltpu.VMEM((1024,), jnp.float32),
        pltpu.VMEM((1024,), jnp.float32),
        pltpu.SemaphoreType.DMA,
    ],
)(_sc_body)


def _loss_body(u_ref, f_ref, o1_ref, o2_ref):
    d0 = u_ref[0, 0] - f_ref[0, 0]
    d1 = u_ref[0, 1] - f_ref[0, 1]
    d2 = u_ref[0, 2] - f_ref[0, 2]
    d3 = u_ref[0, 3] - f_ref[0, 3]
    d4 = u_ref[0, 4] - f_ref[0, 4]
    s1 = jnp.sum(jnp.sqrt(d0 * d0 + d1 * d1 + d2 * d2))
    s2 = jnp.sum(jnp.sqrt(d3 * d3 + d4 * d4))

    @pl.when(pl.program_id(0) == 0)
    def _():
        o1_ref[...] = jnp.zeros((8, 128), jnp.float32)
        o2_ref[...] = jnp.zeros((8, 128), jnp.float32)

    o1_ref[...] += s1
    o2_ref[...] += s2


_loss_kernel = pl.pallas_call(
    _loss_body,
    grid=(B,),
    in_specs=[
        pl.BlockSpec((1, 5, N1 // 128, 128), lambda b: (b, 0, 0, 0)),
        pl.BlockSpec((1, 5, N1 // 128, 128), lambda b: (b, 0, 0, 0)),
    ],
    out_specs=[
        pl.BlockSpec((8, 128), lambda b: (0, 0)),
        pl.BlockSpec((8, 128), lambda b: (0, 0)),
    ],
    out_shape=[
        jax.ShapeDtypeStruct((8, 128), jnp.float32),
        jax.ShapeDtypeStruct((8, 128), jnp.float32),
    ],
)


def kernel(image, output, label, assign_2, assign_4):
    feats = _feat_kernel(image)
    featsf = feats.reshape(B, 5, N1)
    featsr = feats.reshape(B, 5, N1 // 128, 128)
    uf = _sc_kernel(feats.reshape(-1), assign_2.reshape(-1),
                    assign_4.reshape(-1))
    u = uf.reshape(B, 5, N1 // 128, 128)
    s1, s2 = _loss_kernel(u, featsr)
    npix = float(B * N1)
    return s1[0, 0] / npix + 0.1 * (s2[0, 0] / (npix * 16.0))


# speedup vs baseline: 202.1851x; 1.0156x over previous
import functools

import jax
import jax.numpy as jnp
from jax import lax
from jax.experimental import pallas as pl
from jax.experimental.pallas import tpu as pltpu
from jax.experimental.pallas import tpu_sc as plsc

B = 4
H, W = 512, 512
H1, W1 = 256, 256
N1 = H1 * W1
S2 = 128 * 128
S4 = 64 * 64
NC, NS = 2, 16
PPS = N1 // NS
R2 = PPS // 128
R4 = (S2 // NS) // 128

_HIGH = lax.Precision.HIGHEST


def _feat_body(img_ref, out_ref):
    ii = lax.broadcasted_iota(jnp.int32, (H1, H), 0).astype(jnp.float32)
    jj = lax.broadcasted_iota(jnp.int32, (H1, H), 1).astype(jnp.float32)
    w = jnp.maximum(0.0, 1.0 - jnp.abs((jj - 2.0 * ii - 0.5) * 0.5))
    w = w / jnp.sum(w, axis=1, keepdims=True)

    def half(c):
        t = lax.dot(w, c)
        return lax.dot_general(t, w, (((1,), (1,)), ((), ())))

    def to_linear(c):
        big = jnp.exp(2.4 * jnp.log(jnp.maximum((c + 0.055) * (1.0 / 1.055),
                                                1e-12)))
        return jnp.where(c <= 0.04045, c * (1.0 / 12.92), big)

    r = to_linear(half(img_ref[0, 0]))
    g = to_linear(half(img_ref[0, 1]))
    b = to_linear(half(img_ref[0, 2]))

    x = (0.412453 * r + 0.357580 * g + 0.180423 * b) * (1.0 / 0.950456)
    y = 0.212671 * r + 0.715160 * g + 0.072169 * b
    z = (0.019334 * r + 0.119193 * g + 0.950227 * b) * (1.0 / 1.088754)

    d = 6.0 / 29.0
    d3 = d * d * d

    def f(t):
        cbrt = jnp.exp(jnp.log(jnp.maximum(t, d3)) * (1.0 / 3.0))
        return jnp.where(t > d3, cbrt, t / (3.0 * d * d) + 4.0 / 29.0)

    fx, fy, fz = f(x), f(y), f(z)
    out_ref[0, 0] = 116.0 * fy - 16.0
    out_ref[0, 1] = 500.0 * (fx - fy)
    out_ref[0, 2] = 200.0 * (fy - fz)
    out_ref[0, 3] = lax.broadcasted_iota(jnp.int32, (H1, W1), 1).astype(
        jnp.float32)
    out_ref[0, 4] = lax.broadcasted_iota(jnp.int32, (H1, W1), 0).astype(
        jnp.float32)


_feat_kernel = pl.pallas_call(
    _feat_body,
    grid=(B,),
    in_specs=[pl.BlockSpec((1, 3, H, W), lambda b: (b, 0, 0, 0))],
    out_specs=pl.BlockSpec((1, 5, H1, W1), lambda b: (b, 0, 0, 0)),
    out_shape=jax.ShapeDtypeStruct((B, 5, H1, W1), jnp.float32),
)


def _sc_body(featsf, a2f, a4f, u,
             acc2, acc4, m4p, a4t,
             a2c, a4c, a4c2, val25, val45, gbuf, ones, cbuf, dbuf, zbuf, sem):
    cc = lax.axis_index("c")
    ss = lax.axis_index("s")

    @pl.loop(0, 64)
    def _(k):
        zbuf[pl.ds(k * 16, 16)] = jnp.zeros((16,), jnp.float32)

    for k in range(8):
        ones[pl.ds(k * 16, 16)] = jnp.ones((16,), jnp.float32)

    for bb in range(2):
        b = cc * 2 + bb
        for ch in range(6):
            pltpu.sync_copy(zbuf,
                            acc2.at[pl.ds((bb * 6 + ch) * S2 + ss * 1024,
                                          1024)])
            pltpu.sync_copy(zbuf.at[pl.ds(0, 256)],
                            acc4.at[pl.ds((bb * 6 + ch) * S4 + ss * 256,
                                          256)])
        pltpu.sync_copy(a4f.at[pl.ds(b * S2 + ss * 1024, 1024)],
                        a4t.at[pl.ds(bb * S2 + ss * 1024, 1024)])
    plsc.subcore_barrier()

    for bb in range(2):
        b = cc * 2 + bb
        pltpu.sync_copy(a2f.at[pl.ds(b * N1 + ss * PPS, PPS)], a2c.at[bb])
        for ch in range(5):
            pltpu.sync_copy(
                featsf.at[pl.ds((b * 5 + ch) * N1 + ss * PPS, PPS)],
                val25.at[bb, ch])

    for bb in range(2):
        @pl.loop(0, R2)
        def _(j):
            ix = a2c.at[bb, pl.ds(j * 128, 128)]
            pltpu.async_copy(ones, acc2.at[pl.ds(bb * 6 * S2 + 5 * S2,
                                                 S2)].at[ix], sem, add=True)
            for ch in range(5):
                pltpu.async_copy(val25.at[bb, ch, pl.ds(j * 128, 128)],
                                 acc2.at[pl.ds((bb * 6 + ch) * S2,
                                               S2)].at[ix], sem, add=True)

    for _k in range(12):
        pltpu.make_async_copy(featsf.at[pl.ds(0, PPS)],
                              val25.at[0, 0], sem).wait()
    plsc.subcore_barrier()

    for bb in range(2):
        pltpu.sync_copy(acc2.at[pl.ds(bb * 6 * S2 + 5 * S2 + ss * 1024,
                                      1024)], cbuf)
        for ch in range(5):
            o2 = (bb * 6 + ch) * S2 + ss * 1024
            pltpu.sync_copy(acc2.at[pl.ds(o2, 1024)], dbuf)

            @pl.loop(0, 64)
            def _(k):
                o = k * 16
                dbuf[pl.ds(o, 16)] = dbuf[pl.ds(o, 16)] / jnp.maximum(
                    cbuf[pl.ds(o, 16)], 1.0)

            pltpu.sync_copy(dbuf, acc2.at[pl.ds(o2, 1024)])
    plsc.subcore_barrier()

    for bb in range(2):
        b = cc * 2 + bb
        pltpu.sync_copy(a4f.at[pl.ds(b * S2 + ss * 1024, 1024)], a4c.at[bb])
        for ch in range(5):
            pltpu.sync_copy(acc2.at[pl.ds((bb * 6 + ch) * S2 + ss * 1024,
                                          1024)], val45.at[bb, ch])

    for bb in range(2):
        @pl.loop(0, R4)
        def _(j):
            ix = a4c.at[bb, pl.ds(j * 128, 128)]
            pltpu.async_copy(ones, acc4.at[pl.ds(bb * 6 * S4 + 5 * S4,
                                                 S4)].at[ix], sem, add=True)
            for ch in range(5):
                pltpu.async_copy(val45.at[bb, ch, pl.ds(j * 128, 128)],
                                 acc4.at[pl.ds((bb * 6 + ch) * S4,
                                               S4)].at[ix], sem, add=True)

    for _k in range(12):
        pltpu.make_async_copy(featsf.at[pl.ds(0, 1024)],
                              val45.at[0, 0], sem).wait()
    plsc.subcore_barrier()

    for bb in range(2):
        pltpu.sync_copy(acc4.at[pl.ds(bb * 6 * S4 + 5 * S4 + ss * 256,
                                      256)], cbuf.at[pl.ds(0, 256)])
        for ch in range(5):
            pltpu.sync_copy(acc4.at[pl.ds((bb * 6 + ch) * S4 + ss * 256,
                                          256)], dbuf.at[pl.ds(0, 256)])

            @pl.loop(0, 16)
            def _(k):
                o = k * 16
                dbuf[pl.ds(o, 16)] = dbuf[pl.ds(o, 16)] / jnp.maximum(
                    cbuf[pl.ds(o, 16)], 1.0)

            pltpu.sync_copy(dbuf.at[pl.ds(0, 256)],
                            m4p.at[pl.ds((bb * 5 + ch) * S4 + ss * 256,
                                         256)])
    plsc.subcore_barrier()

    for bb in range(2):
        b = cc * 2 + bb
        pltpu.sync_copy(a4t.at[pl.ds(bb * S2, S2)].at[a2c.at[bb]], a4c2)

        for ch in range(5):
            pltpu.async_copy(m4p.at[pl.ds((bb * 5 + ch) * S4,
                                          S4)].at[a4c2], gbuf.at[ch], sem)
        for _k in range(5):
            pltpu.make_async_copy(featsf.at[pl.ds(0, PPS)],
                                  gbuf.at[0], sem).wait()

        for ch in range(5):
            pltpu.async_copy(gbuf.at[ch],
                             u.at[pl.ds((b * 5 + ch) * N1 + ss * PPS, PPS)],
                             sem)
        for _k in range(5):
            pltpu.make_async_copy(featsf.at[pl.ds(0, PPS)],
                                  gbuf.at[0], sem).wait()


_sc_kernel = functools.partial(
    pl.kernel,
    out_type=jax.ShapeDtypeStruct((B * 5 * N1,), jnp.float32),
    mesh=plsc.VectorSubcoreMesh(core_axis_name="c", subcore_axis_name="s",
                                num_cores=NC, num_subcores=NS),
    compiler_params=pltpu.CompilerParams(use_tc_tiling_on_sc=False,
                                         needs_layout_passes=False),
    scratch_types=[
        pltpu.VMEM_SHARED((2 * 6 * S2,), jnp.float32),
        pltpu.VMEM_SHARED((2 * 6 * S4,), jnp.float32),
        pltpu.VMEM_SHARED((2 * 5 * S4,), jnp.float32),
        pltpu.VMEM_SHARED((2 * S2,), jnp.int32),
        pltpu.VMEM((2, PPS), jnp.int32),
        pltpu.VMEM((2, 1024), jnp.int32),
        pltpu.VMEM((PPS,), jnp.int32),
        pltpu.VMEM((2, 5, PPS), jnp.float32),
        pltpu.VMEM((2, 5, 1024), jnp.float32),
        pltpu.VMEM((5, PPS), jnp.float32),
        pltpu.VMEM((128,), jnp.float32),
        pltpu.VMEM((1024,), jnp.float32),
        pltpu.VMEM((1024,), jnp.float32),
        pltpu.VMEM((1024,), jnp.float32),
        pltpu.SemaphoreType.DMA,
    ],
)(_sc_body)


def _loss_body(u_ref, f_ref, o1_ref, o2_ref):
    d0 = u_ref[0, 0] - f_ref[0, 0]
    d1 = u_ref[0, 1] - f_ref[0, 1]
    d2 = u_ref[0, 2] - f_ref[0, 2]
    d3 = u_ref[0, 3] - f_ref[0, 3]
    d4 = u_ref[0, 4] - f_ref[0, 4]
    s1 = jnp.sum(jnp.sqrt(d0 * d0 + d1 * d1 + d2 * d2))
    s2 = jnp.sum(jnp.sqrt(d3 * d3 + d4 * d4))

    @pl.when(pl.program_id(0) == 0)
    def _():
        o1_ref[...] = jnp.zeros((8, 128), jnp.float32)
        o2_ref[...] = jnp.zeros((8, 128), jnp.float32)

    o1_ref[...] += s1
    o2_ref[...] += s2


_loss_kernel = pl.pallas_call(
    _loss_body,
    grid=(B,),
    in_specs=[
        pl.BlockSpec((1, 5, N1 // 128, 128), lambda b: (b, 0, 0, 0)),
        pl.BlockSpec((1, 5, N1 // 128, 128), lambda b: (b, 0, 0, 0)),
    ],
    out_specs=[
        pl.BlockSpec((8, 128), lambda b: (0, 0)),
        pl.BlockSpec((8, 128), lambda b: (0, 0)),
    ],
    out_shape=[
        jax.ShapeDtypeStruct((8, 128), jnp.float32),
        jax.ShapeDtypeStruct((8, 128), jnp.float32),
    ],
)


def kernel(image, output, label, assign_2, assign_4):
    feats = _feat_kernel(image)
    featsf = feats.reshape(B, 5, N1)
    featsr = feats.reshape(B, 5, N1 // 128, 128)
    uf = _sc_kernel(feats.reshape(-1), assign_2.reshape(-1),
                    assign_4.reshape(-1))
    u = uf.reshape(B, 5, N1 // 128, 128)
    s1, s2 = _loss_kernel(u, featsr)
    npix = float(B * N1)
    return s1[0, 0] / npix + 0.1 * (s2[0, 0] / (npix * 16.0))
